# Initial kernel scaffold; baseline (speedup 1.0000x reference)
import jax
import jax.numpy as jnp
from jax.experimental import pallas as pl


def kernel(x, edge_index, edge_attr, W_e1, b_e1, W_n1, b_n1, W_e2, b_e2, W_n2, b_n2):
    raise NotImplementedError("write your pallas kernel here")


# speedup vs baseline: 3.2610x; 3.2610x over previous
import functools

import jax
import jax.numpy as jnp
from jax import lax
from jax.experimental import pallas as pl
from jax.experimental.pallas import tpu as pltpu
from jax.experimental.pallas import tpu_sc as plsc

_N = 10000
_E = 320000
_D = 128
_DE = 16
_ALPHA = 0.5

_B = 128
_K = 8
_SUP = _B * _K
_NTILES = 32
_EPT = 10240
_E_PAD = _NTILES * _EPT
_NROWS = _E_PAD // _B
_ROWS_PER_TILE = _EPT // _B
_NSUP = _EPT // _SUP
_AGG_ROWS = 10240
_ZROWS = _AGG_ROWS // 16


def _sc_edge_body(ps, pd, ep, src2, dst2, enew, aggo,
                  idx_s, idx_d, buf_s, buf_d, buf_e, zbuf, agg_sh, sem):
    c = lax.axis_index("c")
    s = lax.axis_index("s")
    tile = c * 16 + s

    def _zloop(i, _):
        zbuf[i, :] = jnp.zeros((16,), jnp.float32)
        return 0
    lax.fori_loop(0, _ZROWS, _zloop, 0)
    pltpu.sync_copy(zbuf, agg_sh.at[pl.ds(s * _ZROWS, _ZROWS)])
    plsc.subcore_barrier()

    rowbase = tile * _ROWS_PER_TILE

    def _super(g, _):
        r0 = rowbase + g * _K
        pltpu.sync_copy(src2.at[pl.ds(r0, _K)], idx_s)
        pltpu.sync_copy(dst2.at[pl.ds(r0, _K)], idx_d)
        cps = []
        for j in range(_K):
            cps.append(pltpu.async_copy(ps.at[idx_s.at[j]], buf_s.at[j], sem))
            cps.append(pltpu.async_copy(pd.at[idx_d.at[j]], buf_d.at[j], sem))
        cps.append(pltpu.async_copy(ep.at[pl.ds(r0, _K)], buf_e, sem))
        for cp in cps:
            cp.wait()

        for j in range(_K):
            def _cbody(i, _):
                v = buf_s[j, i, :] + buf_d[j, i, :] + buf_e[j, i, :]
                buf_s[j, i, :] = jnp.maximum(v, 0.0)
                return 0
            lax.fori_loop(0, _B, _cbody, 0)

        pltpu.sync_copy(buf_s, enew.at[pl.ds(r0, _K)])
        for j in range(_K):
            pltpu.sync_copy(buf_s.at[j], agg_sh.at[idx_d.at[j]], add=True)
        return 0

    lax.fori_loop(0, _NSUP, _super, 0)
    plsc.subcore_barrier()
    pltpu.sync_copy(agg_sh.at[pl.ds(s * _ZROWS, _ZROWS)],
                    aggo.at[c, pl.ds(s * _ZROWS, _ZROWS)])


@jax.jit
def _sc_edge(ps, pd, ep3, src2, dst2):
    mesh = plsc.VectorSubcoreMesh(core_axis_name="c", subcore_axis_name="s")
    f = pl.kernel(
        _sc_edge_body,
        mesh=mesh,
        compiler_params=pltpu.CompilerParams(use_tc_tiling_on_sc=False),
        out_type=(
            jax.ShapeDtypeStruct((_NROWS, _B, _DE), jnp.float32),
            jax.ShapeDtypeStruct((2, _AGG_ROWS, _DE), jnp.float32),
        ),
        scratch_types=[
            pltpu.VMEM((_K, _B), jnp.int32),
            pltpu.VMEM((_K, _B), jnp.int32),
            pltpu.VMEM((_K, _B, _DE), jnp.float32),
            pltpu.VMEM((_K, _B, _DE), jnp.float32),
            pltpu.VMEM((_K, _B, _DE), jnp.float32),
            pltpu.VMEM((_ZROWS, _DE), jnp.float32),
            pltpu.VMEM_SHARED((_AGG_ROWS, _DE), jnp.float32),
            pltpu.SemaphoreType.DMA,
        ],
    )
    return f(ps, pd, ep3, src2, dst2)


def _node_proj_body(x_ref, ws_ref, wd_ref, be_ref, ps_ref, pd_ref):
    xb = x_ref[...]
    ps_ref[...] = jnp.dot(xb, ws_ref[...], preferred_element_type=jnp.float32)
    pd_ref[...] = (jnp.dot(xb, wd_ref[...], preferred_element_type=jnp.float32)
                   + be_ref[...])


@jax.jit
def _node_proj(x, ws, wd, be):
    bn = 2000
    grid = _N // bn
    return pl.pallas_call(
        _node_proj_body,
        grid=(grid,),
        in_specs=[
            pl.BlockSpec((bn, _D), lambda i: (i, 0)),
            pl.BlockSpec((_D, _DE), lambda i: (0, 0)),
            pl.BlockSpec((_D, _DE), lambda i: (0, 0)),
            pl.BlockSpec((1, _DE), lambda i: (0, 0)),
        ],
        out_specs=[
            pl.BlockSpec((bn, _DE), lambda i: (i, 0)),
            pl.BlockSpec((bn, _DE), lambda i: (i, 0)),
        ],
        out_shape=[
            jax.ShapeDtypeStruct((_N, _DE), jnp.float32),
            jax.ShapeDtypeStruct((_N, _DE), jnp.float32),
        ],
    )(x, ws, wd, be.reshape(1, _DE))


def _edge_proj_body(e_ref, we_ref, ep_ref):
    ep_ref[...] = jnp.dot(e_ref[...], we_ref[...],
                          preferred_element_type=jnp.float32)


@jax.jit
def _edge_proj(e_pad, wee):
    bn = 20480
    grid = _E_PAD // bn
    return pl.pallas_call(
        _edge_proj_body,
        grid=(grid,),
        in_specs=[
            pl.BlockSpec((bn, _DE), lambda i: (i, 0)),
            pl.BlockSpec((_DE, _DE), lambda i: (0, 0)),
        ],
        out_specs=pl.BlockSpec((bn, _DE), lambda i: (i, 0)),
        out_shape=jax.ShapeDtypeStruct((_E_PAD, _DE), jnp.float32),
    )(e_pad, wee)


def _node_update_body(x_ref, a0_ref, a1_ref, wx_ref, wa_ref, bn_ref, out_ref):
    xb = x_ref[...]
    agg = a0_ref[...] + a1_ref[...]
    delta = (jnp.dot(xb, wx_ref[...], preferred_element_type=jnp.float32)
             + jnp.dot(agg, wa_ref[...], preferred_element_type=jnp.float32)
             + bn_ref[...])
    out_ref[...] = _ALPHA * xb + (1.0 - _ALPHA) * jnp.maximum(delta, 0.0)


@jax.jit
def _node_update(x, a0, a1, wx, wa, bnb):
    bn = 2000
    grid = _N // bn
    return pl.pallas_call(
        _node_update_body,
        grid=(grid,),
        in_specs=[
            pl.BlockSpec((bn, _D), lambda i: (i, 0)),
            pl.BlockSpec((bn, _DE), lambda i: (i, 0)),
            pl.BlockSpec((bn, _DE), lambda i: (i, 0)),
            pl.BlockSpec((_D, _D), lambda i: (0, 0)),
            pl.BlockSpec((_DE, _D), lambda i: (0, 0)),
            pl.BlockSpec((1, _D), lambda i: (0, 0)),
        ],
        out_specs=pl.BlockSpec((bn, _D), lambda i: (i, 0)),
        out_shape=jax.ShapeDtypeStruct((_N, _D), jnp.float32),
    )(x, a0, a1, wx, wa, bnb.reshape(1, _D))


def _layer(x, e_pad3, We, be, Wn, bnv, src2, dst2):
    ws = We[:_D]
    wd = We[_D:2 * _D]
    wee = We[2 * _D:]
    ps, pd = _node_proj(x, ws, wd, be)
    ep = _edge_proj(e_pad3.reshape(_E_PAD, _DE), wee)
    enew3, agg = _sc_edge(ps, pd, ep.reshape(_NROWS, _B, _DE), src2, dst2)
    x_new = _node_update(x, agg[0, :_N], agg[1, :_N], Wn[:_D], Wn[_D:], bnv)
    return x_new, enew3


def kernel(x, edge_index, edge_attr, W_e1, b_e1, W_n1, b_n1,
           W_e2, b_e2, W_n2, b_n2):
    src = edge_index[0]
    dst = edge_index[1]
    pad = _E_PAD - _E
    src2 = jnp.concatenate([src, jnp.zeros((pad,), jnp.int32)]).reshape(_NROWS, _B)
    dst2 = jnp.concatenate([dst, jnp.full((pad,), _N, jnp.int32)]).reshape(_NROWS, _B)
    e0_pad3 = jnp.concatenate(
        [edge_attr, jnp.zeros((pad, _DE), jnp.float32)], axis=0
    ).reshape(_NROWS, _B, _DE)

    x1, e1_pad3 = _layer(x, e0_pad3, W_e1, b_e1, W_n1, b_n1, src2, dst2)
    x2, e2_pad3 = _layer(x1, e1_pad3, W_e2, b_e2, W_n2, b_n2, src2, dst2)

    e1 = e1_pad3.reshape(_E_PAD, _DE)[:_E]
    e2 = e2_pad3.reshape(_E_PAD, _DE)[:_E]
    return (x2, e2, (edge_attr, e1, e2))
